# Initial kernel scaffold; baseline (speedup 1.0000x reference)
import jax
import jax.numpy as jnp
from jax.experimental import pallas as pl


def kernel(prob, bx, by, bw, bh, overlap_threshold, randomize_nms_factor, n_objects_max, topk_only):
    raise NotImplementedError("write your pallas kernel here")


# speedup vs baseline: 6.1017x; 6.1017x over previous
import functools

import jax
import jax.numpy as jnp
from jax import lax
from jax.experimental import pallas as pl
from jax.experimental.pallas import tpu as pltpu

_N_ROUNDS = 20
_K = 20
_SCORE_THRESHOLD = 0.3


def _nms_body(bx_ref, by_ref, bw_ref, bh_ref, prob_ref, noise_ref, scal_ref,
              chosen_ref, idx_ref, mask_ref, *, n_real, n_pad):
    f32 = jnp.float32
    bx = bx_ref[0]
    by = by_ref[0]
    bw = bw_ref[0]
    bh = bh_ref[0]
    prob = prob_ref[0]
    noise = noise_ref[0]
    thr = scal_ref[0, 0, 0]
    factor = scal_ref[0, 0, 1]
    topk_only = scal_ref[0, 0, 2]

    score = jnp.maximum(prob + factor * noise, 0.0)

    x1 = bx - 0.5 * bw
    x3 = bx + 0.5 * bw
    y1 = by - 0.5 * bh
    y3 = by + 0.5 * bh
    area = bw * bh

    def to_col(row):
        return jnp.transpose(row, (1, 0))

    x1c, x3c = to_col(x1), to_col(x3)
    y1c, y3c = to_col(y1), to_col(y3)
    areac = to_col(area)

    xi1 = jnp.maximum(x1, x1c)
    yi1 = jnp.maximum(y1, y1c)
    xi3 = jnp.minimum(x3, x3c)
    yi3 = jnp.minimum(y3, y3c)
    inter = jnp.maximum(xi3 - xi1, 0.0) * jnp.maximum(yi3 - yi1, 0.0)
    min_area = jnp.minimum(area, areac)
    overlap = inter / min_area
    mask_ref[...] = (overlap > thr).astype(f32)

    iota_j = lax.broadcasted_iota(jnp.int32, (n_pad, n_pad), 1)
    iota_col = lax.broadcasted_iota(jnp.int32, (n_pad, 1), 0)
    iota_row = lax.broadcasted_iota(jnp.int32, (1, n_pad), 1)

    possible0 = jnp.where(score > _SCORE_THRESHOLD, 1.0, 0.0)
    selected0 = jnp.zeros((n_pad, 1), f32)

    def round_body(_, carry):
        possible_row, selected_col = carry

        def active(args):
            possible_row, selected_col = args
            mask = mask_ref[...]
            v = score * possible_row
            sm = mask * v
            rm = jnp.max(sm, axis=1, keepdims=True)
            cand = jnp.where(sm == rm, iota_j, n_pad)
            am = jnp.min(cand, axis=1, keepdims=True)
            possible_col = to_col(possible_row)
            newly = jnp.where(am == iota_col, possible_col, 0.0)
            selected2 = selected_col + newly
            blocked = jnp.sum(mask * newly, axis=0, keepdims=True)
            possible2 = jnp.where(blocked == 0.0, possible_row, 0.0)
            return possible2, selected2

        any_possible = jnp.sum(possible_row) > 0.0
        return lax.cond(any_possible, active, lambda a: a,
                        (possible_row, selected_col))

    possible_row, selected_col = lax.fori_loop(
        0, _N_ROUNDS, round_body, (possible0, selected0))

    selected_row = jnp.transpose(selected_col, (1, 0))
    chosen = jnp.where(topk_only != 0.0, 1.0, selected_row)
    chosen_ref[0] = chosen

    masked = jnp.where(iota_row < n_real, chosen * score, -1.0)
    idx_vec = jnp.zeros((1, 128), jnp.int32)
    lane128 = lax.broadcasted_iota(jnp.int32, (1, 128), 1)
    for k in range(_K):
        m = jnp.max(masked)
        am = jnp.min(jnp.where(masked == m, iota_row, n_pad))
        idx_vec = jnp.where(lane128 == k, am, idx_vec)
        masked = jnp.where(iota_row == am, -1.0, masked)
    idx_ref[0] = idx_vec


def kernel(prob, bx, by, bw, bh, overlap_threshold, randomize_nms_factor,
           n_objects_max, topk_only):
    n, b = prob.shape[0], prob.shape[1]
    n_pad = ((n + 127) // 128) * 128

    def prep(a, pad_val):
        a2 = jnp.transpose(a[..., 0], (1, 0))
        return jnp.pad(a2, ((0, 0), (0, n_pad - n)),
                       constant_values=pad_val).reshape(b, 1, n_pad)

    bx_p = prep(bx, -100.0)
    by_p = prep(by, -100.0)
    bw_p = prep(bw, 0.0)
    bh_p = prep(bh, 0.0)
    prob_p = prep(prob, 0.0)

    noise = jax.random.normal(jax.random.key(42), (n, b), dtype=jnp.float32)
    noise_p = jnp.pad(noise.T, ((0, 0), (0, n_pad - n))).reshape(b, 1, n_pad)

    scal = jnp.zeros((1, 1, 128), jnp.float32)
    scal = scal.at[0, 0, 0].set(overlap_threshold[0])
    scal = scal.at[0, 0, 1].set(randomize_nms_factor[0])
    scal = scal.at[0, 0, 2].set(jnp.asarray(topk_only).astype(jnp.float32))

    body = functools.partial(_nms_body, n_real=n, n_pad=n_pad)
    chosen_b, idx_b = pl.pallas_call(
        body,
        grid=(b,),
        in_specs=[
            pl.BlockSpec((1, 1, n_pad), lambda i: (i, 0, 0)),
            pl.BlockSpec((1, 1, n_pad), lambda i: (i, 0, 0)),
            pl.BlockSpec((1, 1, n_pad), lambda i: (i, 0, 0)),
            pl.BlockSpec((1, 1, n_pad), lambda i: (i, 0, 0)),
            pl.BlockSpec((1, 1, n_pad), lambda i: (i, 0, 0)),
            pl.BlockSpec((1, 1, n_pad), lambda i: (i, 0, 0)),
            pl.BlockSpec((1, 1, 128), lambda i: (0, 0, 0)),
        ],
        out_specs=[
            pl.BlockSpec((1, 1, n_pad), lambda i: (i, 0, 0)),
            pl.BlockSpec((1, 1, 128), lambda i: (i, 0, 0)),
        ],
        out_shape=[
            jax.ShapeDtypeStruct((b, 1, n_pad), jnp.float32),
            jax.ShapeDtypeStruct((b, 1, 128), jnp.int32),
        ],
        scratch_shapes=[pltpu.VMEM((n_pad, n_pad), jnp.float32)],
    )(bx_p, by_p, bw_p, bh_p, prob_p, noise_p, scal)

    chosen = chosen_b.reshape(b, n_pad)[:, :n].T
    top_k_indices = idx_b.reshape(b, 128)[:, :_K].T
    batch_indices = jnp.broadcast_to(
        jnp.arange(b, dtype=top_k_indices.dtype).reshape(1, -1), (_K, b))
    return chosen, top_k_indices, batch_indices
